# Initial kernel scaffold; baseline (speedup 1.0000x reference)
import jax
import jax.numpy as jnp
from jax.experimental import pallas as pl


def kernel(sparse_rows, sparse_cols, sparse_vals, dense_feat, W, b):
    raise NotImplementedError("write your pallas kernel here")


# speedup vs baseline: 11.7754x; 11.7754x over previous
import functools

import jax
import jax.numpy as jnp
from jax import lax
from jax.experimental import pallas as pl
from jax.experimental.pallas import tpu as pltpu
from jax.experimental.pallas import tpu_sc as plsc

BATCH = 16384
VOCAB = 100000
NNZ = 327680
D = 32
DU = 128

NC = 2
NS = 16
NW = NC * NS
L = 16

CHUNK = 1024
SEG = 128
NSEG = CHUNK // SEG
EPW = NNZ // NW
NCHUNK = EPW // CHUNK
RPS = BATCH // NS


def _sc_body(rows_hbm, cols_hbm, vals_hbm, w_hbm, out_hbm,
             rows_v, cols_v, vals_v, gbuf, acc, sem):
    c = lax.axis_index("c")
    s = lax.axis_index("s")
    wid = c * NS + s

    zero = jnp.zeros((L,), jnp.float32)

    def zero_body(i, carry):
        gbuf[i, pl.ds(0, L)] = zero
        gbuf[i, pl.ds(L, L)] = zero
        return carry

    lax.fori_loop(0, CHUNK, zero_body, 0)
    pltpu.sync_copy(gbuf, acc.at[pl.ds(s * RPS, RPS)])
    plsc.subcore_barrier()

    def chunk_body(k, carry):
        seg_base = wid * (EPW // SEG) + k * NSEG
        pltpu.sync_copy(rows_hbm.at[pl.ds(seg_base, NSEG)], rows_v)
        pltpu.sync_copy(cols_hbm.at[pl.ds(seg_base, NSEG)], cols_v)
        pltpu.sync_copy(vals_hbm.at[pl.ds(wid * EPW + k * CHUNK, CHUNK)],
                        vals_v)
        descs = [
            pltpu.async_copy(w_hbm.at[cols_v.at[j]],
                             gbuf.at[pl.ds(j * SEG, SEG)], sem)
            for j in range(NSEG)
        ]
        for dsc in descs:
            dsc.wait()

        def scale_body(g, carry2):
            v = vals_v[pl.ds(g * L, L)]
            base = g * L
            for i in range(L):
                e = base + i
                val = v[i]
                gbuf[e, pl.ds(0, L)] = gbuf[e, pl.ds(0, L)] * val
                gbuf[e, pl.ds(L, L)] = gbuf[e, pl.ds(L, L)] * val
            return carry2

        lax.fori_loop(0, CHUNK // L, scale_body, 0)

        for j in range(NSEG):
            pltpu.sync_copy(gbuf.at[pl.ds(j * SEG, SEG)],
                            acc.at[rows_v.at[j]], add=True)
        return carry

    lax.fori_loop(0, NCHUNK, chunk_body, 0)
    plsc.subcore_barrier()

    pltpu.sync_copy(acc.at[pl.ds(s * RPS, RPS)],
                    out_hbm.at[c, pl.ds(s * RPS, RPS)])


_sc_accumulate = pl.kernel(
    _sc_body,
    out_type=jax.ShapeDtypeStruct((NC, BATCH, D), jnp.float32),
    mesh=plsc.VectorSubcoreMesh(core_axis_name="c", subcore_axis_name="s"),
    compiler_params=pltpu.CompilerParams(use_tc_tiling_on_sc=False),
    scratch_types=[
        pltpu.VMEM((NSEG, SEG), jnp.int32),
        pltpu.VMEM((NSEG, SEG), jnp.int32),
        pltpu.VMEM((CHUNK,), jnp.float32),
        pltpu.VMEM((CHUNK, D), jnp.float32),
        pltpu.VMEM_SHARED((BATCH, D), jnp.float32),
        pltpu.SemaphoreType.DMA,
    ],
)


_BM = 1024


def _combine_body(p_ref, d_ref, b_ref, o_ref):
    sp = p_ref[0] + p_ref[1] + b_ref[...]
    o_ref[...] = jnp.concatenate([sp, d_ref[...]], axis=-1)


def _tc_combine(partial, dense_feat, b2d):
    return pl.pallas_call(
        _combine_body,
        grid=(BATCH // _BM,),
        in_specs=[
            pl.BlockSpec((NC, _BM, D), lambda i: (0, i, 0)),
            pl.BlockSpec((_BM, DU), lambda i: (i, 0)),
            pl.BlockSpec((1, D), lambda i: (0, 0)),
        ],
        out_specs=pl.BlockSpec((_BM, D + DU), lambda i: (i, 0)),
        out_shape=jax.ShapeDtypeStruct((BATCH, D + DU), jnp.float32),
    )(partial, dense_feat, b2d)


def kernel(sparse_rows, sparse_cols, sparse_vals, dense_feat, W, b):
    rows2d = sparse_rows.astype(jnp.int32).reshape(NNZ // SEG, SEG)
    cols2d = sparse_cols.astype(jnp.int32).reshape(NNZ // SEG, SEG)
    partial = _sc_accumulate(rows2d, cols2d, sparse_vals, W)
    return _tc_combine(partial, dense_feat, b.reshape(1, D))


# speedup vs baseline: 14.4307x; 1.2255x over previous
import functools

import jax
import jax.numpy as jnp
from jax import lax
from jax.experimental import pallas as pl
from jax.experimental.pallas import tpu as pltpu
from jax.experimental.pallas import tpu_sc as plsc

BATCH = 16384
VOCAB = 100000
NNZ = 327680
D = 32
DU = 128

NC = 2
NS = 16
NW = NC * NS
L = 16

CHUNK = 1024
SEG = 128
NSEG = CHUNK // SEG
EPW = NNZ // NW
NCHUNK = EPW // CHUNK
SEGPW = EPW // SEG
RPS = BATCH // NS


def _sc_body(rows_hbm, cols_hbm, vals_hbm, w_hbm, out_hbm,
             rows_all, cols_all, vals_all, gbuf0, gbuf1, acc,
             sem_i, sem_g0, sem_g1, sem_s0, sem_s1):
    c = lax.axis_index("c")
    s = lax.axis_index("s")
    wid = c * NS + s
    seg0 = wid * SEGPW

    idx_descs = [
        pltpu.async_copy(rows_hbm.at[pl.ds(seg0, SEGPW)], rows_all, sem_i),
        pltpu.async_copy(cols_hbm.at[pl.ds(seg0, SEGPW)], cols_all, sem_i),
        pltpu.async_copy(vals_hbm.at[pl.ds(seg0, SEGPW)], vals_all, sem_i),
    ]

    zero = jnp.zeros((L,), jnp.float32)

    def zero_body(i, carry):
        gbuf0[i, pl.ds(0, L)] = zero
        gbuf0[i, pl.ds(L, L)] = zero
        return carry

    lax.fori_loop(0, CHUNK, zero_body, 0)
    pltpu.sync_copy(gbuf0, acc.at[pl.ds(s * RPS, RPS)])
    plsc.subcore_barrier()
    for dsc in idx_descs:
        dsc.wait()

    gbufs = (gbuf0, gbuf1)
    sems_g = (sem_g0, sem_g1)
    sems_s = (sem_s0, sem_s1)

    def fire_gather(k):
        gb, sem = gbufs[k % 2], sems_g[k % 2]
        return [
            pltpu.async_copy(w_hbm.at[cols_all.at[k * NSEG + j]],
                             gb.at[pl.ds(j * SEG, SEG)], sem)
            for j in range(NSEG)
        ]

    descs_g = {0: fire_gather(0)}
    descs_s = {}
    for k in range(NCHUNK):
        gb = gbufs[k % 2]
        for dsc in descs_g.pop(k):
            dsc.wait()
        if k + 1 < NCHUNK:
            if k >= 1:
                for dsc in descs_s.pop(k - 1):
                    dsc.wait()
            descs_g[k + 1] = fire_gather(k + 1)

        def scale_body(g, carry, gb=gb, k=k):
            v = vals_all[k * NSEG + g // 8, pl.ds((g % 8) * L, L)]
            for i in range(L):
                e = g * L + i
                val = v[i]
                gb[e, pl.ds(0, L)] = gb[e, pl.ds(0, L)] * val
                gb[e, pl.ds(L, L)] = gb[e, pl.ds(L, L)] * val
            return carry

        lax.fori_loop(0, CHUNK // L, scale_body, 0)

        descs_s[k] = [
            pltpu.async_copy(gb.at[pl.ds(j * SEG, SEG)],
                             acc.at[rows_all.at[k * NSEG + j]],
                             sems_s[k % 2], add=True)
            for j in range(NSEG)
        ]
    for k in (NCHUNK - 2, NCHUNK - 1):
        for dsc in descs_s.pop(k):
            dsc.wait()
    plsc.subcore_barrier()

    pltpu.sync_copy(acc.at[pl.ds(s * RPS, RPS)],
                    out_hbm.at[c, pl.ds(s * RPS, RPS)])


_sc_accumulate = pl.kernel(
    _sc_body,
    out_type=jax.ShapeDtypeStruct((NC, BATCH, D), jnp.float32),
    mesh=plsc.VectorSubcoreMesh(core_axis_name="c", subcore_axis_name="s"),
    compiler_params=pltpu.CompilerParams(use_tc_tiling_on_sc=False),
    scratch_types=[
        pltpu.VMEM((SEGPW, SEG), jnp.int32),
        pltpu.VMEM((SEGPW, SEG), jnp.int32),
        pltpu.VMEM((SEGPW, SEG), jnp.float32),
        pltpu.VMEM((CHUNK, D), jnp.float32),
        pltpu.VMEM((CHUNK, D), jnp.float32),
        pltpu.VMEM_SHARED((BATCH, D), jnp.float32),
        pltpu.SemaphoreType.DMA,
        pltpu.SemaphoreType.DMA,
        pltpu.SemaphoreType.DMA,
        pltpu.SemaphoreType.DMA,
        pltpu.SemaphoreType.DMA,
    ],
)


_BM = 1024


def _combine_body(p_ref, d_ref, b_ref, o_ref):
    sp = p_ref[0] + p_ref[1] + b_ref[...]
    o_ref[...] = jnp.concatenate([sp, d_ref[...]], axis=-1)


def _tc_combine(partial, dense_feat, b2d):
    return pl.pallas_call(
        _combine_body,
        grid=(BATCH // _BM,),
        in_specs=[
            pl.BlockSpec((NC, _BM, D), lambda i: (0, i, 0)),
            pl.BlockSpec((_BM, DU), lambda i: (i, 0)),
            pl.BlockSpec((1, D), lambda i: (0, 0)),
        ],
        out_specs=pl.BlockSpec((_BM, D + DU), lambda i: (i, 0)),
        out_shape=jax.ShapeDtypeStruct((BATCH, D + DU), jnp.float32),
    )(partial, dense_feat, b2d)


def kernel(sparse_rows, sparse_cols, sparse_vals, dense_feat, W, b):
    rows2d = sparse_rows.astype(jnp.int32).reshape(NNZ // SEG, SEG)
    cols2d = sparse_cols.astype(jnp.int32).reshape(NNZ // SEG, SEG)
    vals2d = sparse_vals.reshape(NNZ // SEG, SEG)
    partial = _sc_accumulate(rows2d, cols2d, vals2d, W)
    return _tc_combine(partial, dense_feat, b.reshape(1, D))
